# baseline (speedup 1.0000x reference)
import functools

import jax
import jax.numpy as jnp
from jax import lax
from jax.experimental import pallas as pl
from jax.experimental.pallas import tpu as pltpu
from jax.experimental.pallas import tpu_sc as plsc

B = 16384
N_USER = 1000000
D_U = 32
N_CAT = 1000
D_C = 16
D_OUT = D_U + D_C + 2

NC = 2
NS = 16
NW = NC * NS
BPW = B // NW
L = 16


def _sc_body(uid_hbm, cid_hbm, wu_hbm, wc_hbm,
             outu_hbm, outc_hbm,
             uid_v, cid_v, wc_v, outc_v, sem_u, sem_c):
    wid = lax.axis_index("s") * NC + lax.axis_index("c")
    base = wid * BPW

    pltpu.sync_copy(uid_hbm.at[pl.ds(base, BPW)], uid_v)
    pltpu.sync_copy(cid_hbm.at[pl.ds(base, BPW)], cid_v)
    cat_stage = pltpu.async_copy(wc_hbm, wc_v, sem_c)

    def user_body(g, carry):
        u16 = uid_v[pl.ds(g * L, L)]
        for r in range(L):
            b = g * L + r
            pltpu.async_copy(wu_hbm.at[pl.ds(u16[r], 1)],
                             outu_hbm.at[pl.ds(base + b, 1)], sem_u)
        return carry

    lax.fori_loop(0, BPW // L, user_body, 0, unroll=2)

    cat_stage.wait()

    def cat_body(g, carry):
        c16 = cid_v[pl.ds(g * L, L)] * D_C
        for r in range(L):
            b = g * L + r
            outc_v[pl.ds(b * D_C, D_C)] = wc_v[pl.ds(c16[r], D_C)]
        return carry

    lax.fori_loop(0, BPW // L, cat_body, 0, unroll=2)

    pltpu.sync_copy(outc_v, outc_hbm.at[pl.ds(base * D_C, BPW * D_C)])

    pltpu.make_async_copy(wu_hbm.at[pl.ds(0, BPW)],
                          outu_hbm.at[pl.ds(base, BPW)], sem_u).wait()


def kernel(user_id, category, day_sin, day_cos, W_user, W_category):
    mesh = plsc.VectorSubcoreMesh(core_axis_name="c", subcore_axis_name="s")
    run = pl.kernel(
        _sc_body, mesh=mesh,
        compiler_params=pltpu.CompilerParams(needs_layout_passes=False),
        out_type=(jax.ShapeDtypeStruct((B, D_U), jnp.float32),
                  jax.ShapeDtypeStruct((B * D_C,), jnp.float32)),
        scratch_types=[
            pltpu.VMEM((BPW,), jnp.int32),
            pltpu.VMEM((BPW,), jnp.int32),
            pltpu.VMEM((N_CAT * D_C,), jnp.float32),
            pltpu.VMEM((BPW * D_C,), jnp.float32),
            pltpu.SemaphoreType.DMA,
            pltpu.SemaphoreType.DMA,
        ],
    )
    eu, ec = run(user_id, category, W_user, W_category.reshape(N_CAT * D_C))
    return jnp.concatenate([eu, ec.reshape(B, D_C), day_sin, day_cos], axis=1)


# speedup vs baseline: 1.7340x; 1.7340x over previous
import functools

import jax
import jax.numpy as jnp
from jax import lax
from jax.experimental import pallas as pl
from jax.experimental.pallas import tpu as pltpu
from jax.experimental.pallas import tpu_sc as plsc

B = 16384
N_USER = 1000000
D_U = 32
N_CAT = 1000
D_C = 16
D_OUT = D_U + D_C + 2

NC = 2
NS = 16
NW = NC * NS
BPW = B // NW
L = 16


def _sc_body(uid_hbm, cid_hbm, wu_hbm, wc_hbm,
             outu_hbm, outc_hbm,
             uid_v, cid_v, wc_v, outc_v, rows_v, sem_u, sem_c):
    wid = lax.axis_index("s") * NC + lax.axis_index("c")
    base = wid * BPW

    pltpu.sync_copy(uid_hbm.at[pl.ds(base, BPW)], uid_v)
    pltpu.sync_copy(cid_hbm.at[pl.ds(base, BPW)], cid_v)
    cat_stage = pltpu.async_copy(wc_hbm, wc_v, sem_c)

    def user_body(g, carry):
        u16 = uid_v[pl.ds(g * L, L)]
        for r in range(L):
            b = g * L + r
            pltpu.async_copy(wu_hbm.at[pl.ds(u16[r], 1)],
                             rows_v.at[pl.ds(b, 1)], sem_u)
        return carry

    lax.fori_loop(0, BPW // L, user_body, 0, unroll=2)

    cat_stage.wait()

    def cat_body(g, carry):
        c16 = cid_v[pl.ds(g * L, L)] * D_C
        for r in range(L):
            b = g * L + r
            outc_v[pl.ds(b * D_C, D_C)] = wc_v[pl.ds(c16[r], D_C)]
        return carry

    lax.fori_loop(0, BPW // L, cat_body, 0, unroll=2)

    pltpu.sync_copy(outc_v, outc_hbm.at[pl.ds(base * D_C, BPW * D_C)])

    pltpu.make_async_copy(wu_hbm.at[pl.ds(0, BPW)], rows_v, sem_u).wait()
    pltpu.sync_copy(rows_v, outu_hbm.at[pl.ds(base, BPW)])


def kernel(user_id, category, day_sin, day_cos, W_user, W_category):
    mesh = plsc.VectorSubcoreMesh(core_axis_name="c", subcore_axis_name="s")
    run = pl.kernel(
        _sc_body, mesh=mesh,
        compiler_params=pltpu.CompilerParams(needs_layout_passes=False),
        out_type=(jax.ShapeDtypeStruct((B, D_U), jnp.float32),
                  jax.ShapeDtypeStruct((B * D_C,), jnp.float32)),
        scratch_types=[
            pltpu.VMEM((BPW,), jnp.int32),
            pltpu.VMEM((BPW,), jnp.int32),
            pltpu.VMEM((N_CAT * D_C,), jnp.float32),
            pltpu.VMEM((BPW * D_C,), jnp.float32),
            pltpu.VMEM((BPW, D_U), jnp.float32),
            pltpu.SemaphoreType.DMA,
            pltpu.SemaphoreType.DMA,
        ],
    )
    eu, ec = run(user_id, category, W_user, W_category.reshape(N_CAT * D_C))
    return jnp.concatenate([eu, ec.reshape(B, D_C), day_sin, day_cos], axis=1)


# speedup vs baseline: 1.7341x; 1.0000x over previous
import functools

import jax
import jax.numpy as jnp
from jax import lax
from jax.experimental import pallas as pl
from jax.experimental.pallas import tpu as pltpu
from jax.experimental.pallas import tpu_sc as plsc

B = 16384
N_USER = 1000000
D_U = 32
N_CAT = 1000
D_C = 16
D_OUT = D_U + D_C + 2

NC = 2
NS = 16
NW = NC * NS
BPW = B // NW
L = 16


def _sc_body(uid_hbm, cid_hbm, wu_hbm, wc_hbm,
             outu_hbm, outc_hbm,
             uid_v, cid_v, wc_v, outc_v, rows_v, sem_u, sem_c):
    wid = lax.axis_index("s") * NC + lax.axis_index("c")
    base = wid * BPW

    pltpu.sync_copy(uid_hbm.at[pl.ds(base, BPW)], uid_v)
    pltpu.sync_copy(cid_hbm.at[pl.ds(base, BPW)], cid_v)
    cat_stage = pltpu.async_copy(wc_hbm, wc_v, sem_c)

    def user_body(g, carry):
        u16 = uid_v[pl.ds(g * L, L)]
        for r in range(L):
            b = g * L + r
            pltpu.async_copy(wu_hbm.at[pl.ds(u16[r], 1)],
                             rows_v.at[pl.ds(b, 1)], sem_u)
        return carry

    lax.fori_loop(0, BPW // L, user_body, 0, unroll=4)

    cat_stage.wait()

    def cat_body(g, carry):
        c16 = cid_v[pl.ds(g * L, L)] * D_C
        for r in range(L):
            b = g * L + r
            outc_v[pl.ds(b * D_C, D_C)] = wc_v[pl.ds(c16[r], D_C)]
        return carry

    lax.fori_loop(0, BPW // L, cat_body, 0, unroll=2)

    pltpu.sync_copy(outc_v, outc_hbm.at[pl.ds(base * D_C, BPW * D_C)])

    pltpu.make_async_copy(wu_hbm.at[pl.ds(0, BPW)], rows_v, sem_u).wait()
    pltpu.sync_copy(rows_v, outu_hbm.at[pl.ds(base, BPW)])


def kernel(user_id, category, day_sin, day_cos, W_user, W_category):
    mesh = plsc.VectorSubcoreMesh(core_axis_name="c", subcore_axis_name="s")
    run = pl.kernel(
        _sc_body, mesh=mesh,
        compiler_params=pltpu.CompilerParams(needs_layout_passes=False),
        out_type=(jax.ShapeDtypeStruct((B, D_U), jnp.float32),
                  jax.ShapeDtypeStruct((B * D_C,), jnp.float32)),
        scratch_types=[
            pltpu.VMEM((BPW,), jnp.int32),
            pltpu.VMEM((BPW,), jnp.int32),
            pltpu.VMEM((N_CAT * D_C,), jnp.float32),
            pltpu.VMEM((BPW * D_C,), jnp.float32),
            pltpu.VMEM((BPW, D_U), jnp.float32),
            pltpu.SemaphoreType.DMA,
            pltpu.SemaphoreType.DMA,
        ],
    )
    eu, ec = run(user_id, category, W_user, W_category.reshape(N_CAT * D_C))
    return jnp.concatenate([eu, ec.reshape(B, D_C), day_sin, day_cos], axis=1)
